# Initial kernel scaffold; baseline (speedup 1.0000x reference)
import jax
import jax.numpy as jnp
from jax.experimental import pallas as pl


def kernel(image_resnet, params, ref_vertices, edge_index, edge_weight):
    raise NotImplementedError("write your pallas kernel here")


# speedup vs baseline: 1.0032x; 1.0032x over previous
import jax
import jax.numpy as jnp
from jax.experimental import pallas as pl


def _graph_linear(x, W, b):
    return jnp.einsum('oc,bcn->bon', W, x) + b[None, :, None]


def _group_norm(x, gamma, beta, groups, eps=1e-5):
    B, C, N = x.shape
    xg = x.reshape(B, groups, C // groups, N)
    mean = xg.mean(axis=(2, 3), keepdims=True)
    var = xg.var(axis=(2, 3), keepdims=True)
    xg = (xg - mean) / jnp.sqrt(var + eps)
    return xg.reshape(B, C, N) * gamma[None, :, None] + beta[None, :, None]


def _spmm(h, edge_index, edge_weight, n):
    src = edge_index[0]
    dst = edge_index[1]
    msgs = h[:, src, :] * edge_weight[None, :, None]
    out = jax.ops.segment_sum(jnp.transpose(msgs, (1, 0, 2)), dst, num_segments=n)
    return jnp.transpose(out, (1, 0, 2))


def _res_block(x, p, edge_index, edge_weight):
    in_ch = x.shape[1]
    half = p['lin1_W'].shape[0]
    y = jax.nn.relu(_group_norm(x, p['pre_g'], p['pre_b'], in_ch // 8))
    y = _graph_linear(y, p['lin1_W'], p['lin1_b'])
    y = jax.nn.relu(_group_norm(y, p['n1_g'], p['n1_b'], half // 8))
    yt = jnp.transpose(y, (0, 2, 1))
    yt = _spmm(jnp.matmul(yt, p['conv_W']), edge_index, edge_weight, yt.shape[1]) + p['conv_b']
    y = jnp.transpose(yt, (0, 2, 1))
    y = jax.nn.relu(_group_norm(y, p['n2_g'], p['n2_b'], half // 8))
    y = _graph_linear(y, p['lin2_W'], p['lin2_b'])
    if 'skip_W' in p:
        x = _graph_linear(x, p['skip_W'], p['skip_b'])
    return x + y


def _identity_pallas(x):
    def body(x_ref, o_ref):
        o_ref[...] = x_ref[...]
    return pl.pallas_call(
        body, out_shape=jax.ShapeDtypeStruct(x.shape, x.dtype))(x)


def kernel(image_resnet, params, ref_vertices, edge_index, edge_weight):
    B = image_resnet.shape[0]
    N = ref_vertices.shape[1]
    W0 = params['lin0_W']
    y_rv = jnp.einsum('oc,cn->on', W0[:, :3], ref_vertices)
    y_enc = jnp.einsum('oc,bc->bo', W0[:, 3:], image_resnet)
    y = y_rv[None] + y_enc[:, :, None] + params['lin0_b'][None, :, None]
    for p in params['blocks']:
        y = _res_block(y, p, edge_index, edge_weight)
    s = _graph_linear(y, params['shape_W1'], params['shape_b1'])
    s = jax.nn.relu(s)
    s = _graph_linear(s, params['shape_W2'], params['shape_b2'])
    s = _group_norm(s, params['shape_ng'], params['shape_nb'], 4)
    s = jax.nn.relu(s)
    s = _graph_linear(s, params['shape_W3'], params['shape_b3'])
    return _identity_pallas(s)


# speedup vs baseline: 7.2474x; 7.2239x over previous
import functools
import jax
import jax.numpy as jnp
from jax import lax
from jax.experimental import pallas as pl
from jax.experimental.pallas import tpu as pltpu
from jax.experimental.pallas import tpu_sc as plsc

N_VERTS = 6890
NPAD = 6912
BATCH = 4
CHALF = 256
ROW = BATCH * CHALF
NCHUNK = 8
CW = ROW // NCHUNK
NSC = 2
NSUB = 16
G = 128
RPT = NPAD // NSUB


def _spmm_sc(h4, src4, dstv, w16, n_batches):
    epad = dstv.shape[0]
    ept = epad // NSUB

    mesh = plsc.VectorSubcoreMesh(core_axis_name="c", subcore_axis_name="s",
                                  num_cores=NSC, num_subcores=NSUB)

    @functools.partial(
        pl.kernel, mesh=mesh,
        out_type=jax.ShapeDtypeStruct((NPAD, ROW), jnp.float32),
        scratch_types=[
            pltpu.VMEM((G,), jnp.int32),
            pltpu.VMEM((G,), jnp.int32),
            pltpu.VMEM((G, 16), jnp.float32),
            pltpu.VMEM((G, CW), jnp.float32),
            pltpu.VMEM((16, CW), jnp.float32),
            pltpu.VMEM_SHARED((NPAD, CW), jnp.float32),
            pltpu.SemaphoreType.DMA,
        ],
    )
    def k(h4_hbm, src4_hbm, dstv_hbm, w16_hbm, out_hbm,
          idx_v, dst_v, w_v, rows_v, zero_v, acc, sem):
        core = lax.axis_index("c")
        sub = lax.axis_index("s")
        zvec = jnp.zeros((16,), jnp.float32)
        for r in range(16):
            for kk in range(CW // 16):
                zero_v[r, pl.ds(kk * 16, 16)] = zvec
        for cc in range(NCHUNK // NSC):
            chunk = core * (NCHUNK // NSC) + cc
            for z in range(RPT // 16):
                pltpu.sync_copy(zero_v, acc.at[pl.ds(sub * RPT + z * 16, 16)])
            plsc.subcore_barrier()

            def batch_body(b, _):
                off = sub * ept + b * G
                pltpu.sync_copy(src4_hbm.at[pl.ds(chunk * epad + off, G)], idx_v)
                pltpu.sync_copy(dstv_hbm.at[pl.ds(off, G)], dst_v)
                pltpu.sync_copy(w16_hbm.at[pl.ds(off, G)], w_v)
                pltpu.async_copy(h4_hbm.at[idx_v], rows_v, sem).wait()

                def g_body(g, _):
                    wv = w_v[g, :]
                    for kk in range(CW // 16):
                        sl = pl.ds(kk * 16, 16)
                        rows_v[g, sl] = rows_v[g, sl] * wv
                    return 0

                lax.fori_loop(0, G, g_body, 0)
                pltpu.sync_copy(rows_v, acc.at[dst_v], add=True)
                return 0

            lax.fori_loop(0, n_batches, batch_body, 0)
            plsc.subcore_barrier()
            pltpu.sync_copy(
                acc.at[pl.ds(sub * RPT, RPT)],
                out_hbm.at[pl.ds(sub * RPT, RPT), pl.ds(chunk * CW, CW)])
            plsc.subcore_barrier()

    return k(h4, src4, dstv, w16)


def _spmm(h, edge_index, edge_weight, n):
    B, N, C = h.shape
    src = edge_index[0]
    dst = edge_index[1]
    E = src.shape[0]
    epad = ((E + NSUB * G - 1) // (NSUB * G)) * (NSUB * G)
    n_batches = epad // (NSUB * G)
    pad = epad - E
    src_p = jnp.pad(src, (0, pad))
    dst_p = jnp.pad(dst, (0, pad))
    w_p = jnp.pad(edge_weight, (0, pad))
    src4 = (src_p[None, :] * NCHUNK
            + jnp.arange(NCHUNK, dtype=jnp.int32)[:, None]).reshape(-1)
    w16 = jnp.broadcast_to(w_p[:, None], (epad, 16))
    h_vm = jnp.transpose(h, (1, 0, 2)).reshape(N, B * C)
    h_vm = jnp.pad(h_vm, ((0, NPAD - N), (0, 0)))
    h4 = h_vm.reshape(NPAD * NCHUNK, CW)
    out = _spmm_sc(h4, src4, dst_p, w16, n_batches)
    return jnp.transpose(out[:N].reshape(N, B, C), (1, 0, 2))


def _graph_linear(x, W, b):
    return jnp.einsum('oc,bcn->bon', W, x) + b[None, :, None]


def _group_norm(x, gamma, beta, groups, eps=1e-5):
    B, C, N = x.shape
    xg = x.reshape(B, groups, C // groups, N)
    mean = xg.mean(axis=(2, 3), keepdims=True)
    var = xg.var(axis=(2, 3), keepdims=True)
    xg = (xg - mean) / jnp.sqrt(var + eps)
    return xg.reshape(B, C, N) * gamma[None, :, None] + beta[None, :, None]


def _res_block(x, p, edge_index, edge_weight):
    in_ch = x.shape[1]
    half = p['lin1_W'].shape[0]
    y = jax.nn.relu(_group_norm(x, p['pre_g'], p['pre_b'], in_ch // 8))
    y = _graph_linear(y, p['lin1_W'], p['lin1_b'])
    y = jax.nn.relu(_group_norm(y, p['n1_g'], p['n1_b'], half // 8))
    yt = jnp.transpose(y, (0, 2, 1))
    yt = _spmm(jnp.matmul(yt, p['conv_W']), edge_index, edge_weight, yt.shape[1]) + p['conv_b']
    y = jnp.transpose(yt, (0, 2, 1))
    y = jax.nn.relu(_group_norm(y, p['n2_g'], p['n2_b'], half // 8))
    y = _graph_linear(y, p['lin2_W'], p['lin2_b'])
    if 'skip_W' in p:
        x = _graph_linear(x, p['skip_W'], p['skip_b'])
    return x + y


def kernel(image_resnet, params, ref_vertices, edge_index, edge_weight):
    B = image_resnet.shape[0]
    N = ref_vertices.shape[1]
    W0 = params['lin0_W']
    y_rv = jnp.einsum('oc,cn->on', W0[:, :3], ref_vertices)
    y_enc = jnp.einsum('oc,bc->bo', W0[:, 3:], image_resnet)
    y = y_rv[None] + y_enc[:, :, None] + params['lin0_b'][None, :, None]
    for p in params['blocks']:
        y = _res_block(y, p, edge_index, edge_weight)
    s = _graph_linear(y, params['shape_W1'], params['shape_b1'])
    s = jax.nn.relu(s)
    s = _graph_linear(s, params['shape_W2'], params['shape_b2'])
    s = _group_norm(s, params['shape_ng'], params['shape_nb'], 4)
    s = jax.nn.relu(s)
    s = _graph_linear(s, params['shape_W3'], params['shape_b3'])
    return s
